# baseline (speedup 1.0000x reference)
import functools

import jax
import jax.numpy as jnp
from jax import lax
from jax.experimental import pallas as pl
from jax.experimental.pallas import tpu as pltpu
from jax.experimental.pallas import tpu_sc as plsc

N = 50000
E = 800000
D0 = 128
D1 = 64
NMASK = 10000
FEAT_W = 8
DISC_W = 6

NC = 2
NS = 16
NW = NC * NS

CHUNK = 128
NCHUNKS = E // CHUNK
ETRIPS = (NCHUNKS + NW - 1) // NW
ROWS_PER_SUB = 3128
N_PAD = NS * ROWS_PER_SUB
MCHUNK = 16
PKR = CHUNK // 8
NMCHUNKS = NMASK // MCHUNK
MTRIPS = (NMCHUNKS + NW - 1) // NW

def _mesh():
    return plsc.VectorSubcoreMesh(
        core_axis_name="c", subcore_axis_name="s",
        num_cores=NC, num_subcores=NS)


FROWS = N_PAD // 16

_PERM_E = ([D0 + 10 + j for j in range(8)] + [D0 + 18 + j for j in range(6)]
           + [D0 + 8, D0 + 9])
_PERM_D = ([D1 + 10 + j for j in range(8)] + [D1 + 18 + j for j in range(6)]
           + [D1 + 8, D1 + 9])


def _pipe_helpers(wid, sL, sS, acc, dstv, datv, loads):
    def _valid(t):
        return (t >= 0) & ((wid + NW * t) < NCHUNKS)

    def _off(t):
        return (wid + NW * t) * CHUNK

    def _issue_loads(t, b):
        @pl.when(_valid(t))
        def _():
            off = _off(t)
            for h, v in loads:
                pltpu.async_copy(h.at[pl.ds(off, CHUNK)], v.at[b], sL[b])

    def _wait_loads(t, b):
        @pl.when(_valid(t))
        def _():
            off = _off(t)
            for h, v in loads:
                pltpu.make_async_copy(h.at[pl.ds(off, CHUNK)], v.at[b],
                                      sL[b]).wait()

    def _wait_scatters(t, b):
        @pl.when(_valid(t))
        def _():
            for a, v in zip(acc, datv):
                pltpu.make_async_copy(v.at[b], a.at[dstv.at[b]], sS[b]).wait()

    return _valid, _issue_loads, _wait_loads, _wait_scatters


@functools.cache
def _build_sc_attr_scatter():
    @functools.partial(
        pl.kernel,
        out_type=(
            jax.ShapeDtypeStruct((NC, N_PAD, 16), jnp.float32),
            jax.ShapeDtypeStruct((FROWS, 16), jnp.float32),
        ),
        mesh=_mesh(),
        compiler_params=pltpu.CompilerParams(
            use_tc_tiling_on_sc=False, needs_layout_passes=False),
        scratch_types=[
            pltpu.VMEM((2, CHUNK), jnp.int32),
            pltpu.VMEM((2, CHUNK, 16), jnp.float32),
            pltpu.VMEM((NMASK,), jnp.int32),
            pltpu.VMEM((ROWS_PER_SUB, 16), jnp.float32),
            pltpu.SemaphoreType.DMA,
            pltpu.SemaphoreType.DMA,
            pltpu.SemaphoreType.DMA,
            pltpu.SemaphoreType.DMA,
            pltpu.VMEM_SHARED((N_PAD, 16), jnp.float32),
        ],
    )
    def sc_attr_scatter(dst_h, feat_h, dda_h, mask_h, zA_h, accA_o, flag_o,
                        dstv, attrv, midxv, zb16, sL0, sL1, sS0, sS1, accA):
        c = lax.axis_index("c")
        s = lax.axis_index("s")
        wid = c * NS + s
        rs = s * ROWS_PER_SUB
        sL = (sL0, sL1)
        sS = (sS0, sS1)

        pltpu.sync_copy(zA_h, accA.at[pl.ds(rs, ROWS_PER_SUB)])

        @pl.when(wid == 0)
        def _():
            pltpu.sync_copy(zA_h, zb16)
            pltpu.sync_copy(mask_h, midxv)
            ones = jnp.full((MCHUNK,), 1.0, jnp.float32)

            def fbody(t, carry):
                mi = midxv[pl.ds(t * MCHUNK, MCHUNK)]
                plsc.store_scatter(zb16, [mi >> 4, mi & 15], ones)
                return carry

            lax.fori_loop(0, NMCHUNKS, fbody, 0)
            pltpu.sync_copy(zb16.at[pl.ds(0, FROWS)], flag_o)

        plsc.subcore_barrier()

        def valid(t):
            return (t >= 0) & ((wid + NW * t) < NCHUNKS)

        def issue_loads(t, b):
            @pl.when(valid(t))
            def _():
                ci = wid + NW * t
                pltpu.async_copy(dst_h.at[pl.ds(ci * CHUNK, CHUNK)],
                                 dstv.at[b], sL[b])
                pltpu.async_copy(feat_h.at[pl.ds(ci * CHUNK, CHUNK)],
                                 attrv.at[b, :, pl.ds(0, 8)], sL[b])
                pltpu.async_copy(dda_h.at[pl.ds(ci * CHUNK, CHUNK)],
                                 attrv.at[b, :, pl.ds(8, 8)], sL[b])

        def wait_loads(t, b):
            @pl.when(valid(t))
            def _():
                ci = wid + NW * t
                pltpu.make_async_copy(dst_h.at[pl.ds(ci * CHUNK, CHUNK)],
                                      dstv.at[b], sL[b]).wait()
                pltpu.make_async_copy(feat_h.at[pl.ds(ci * CHUNK, CHUNK)],
                                      attrv.at[b, :, pl.ds(0, 8)],
                                      sL[b]).wait()
                pltpu.make_async_copy(dda_h.at[pl.ds(ci * CHUNK, CHUNK)],
                                      attrv.at[b, :, pl.ds(8, 8)],
                                      sL[b]).wait()

        def wait_scatters(t, b):
            @pl.when(valid(t))
            def _():
                pltpu.make_async_copy(attrv.at[b], accA.at[dstv.at[b]],
                                      sS[b]).wait()

        def step(t, b):
            bn = 1 - b
            wait_scatters(t - 1, bn)
            issue_loads(t + 1, bn)

            @pl.when(valid(t))
            def _():
                wait_loads(t, b)
                pltpu.async_copy(attrv.at[b], accA.at[dstv.at[b]], sS[b],
                                 add=True)

        issue_loads(0, 0)

        def ebody(j, carry):
            step(2 * j, 0)
            step(2 * j + 1, 1)
            return carry

        lax.fori_loop(0, ETRIPS // 2, ebody, 0)
        wait_scatters(ETRIPS - 1, 1)

        plsc.subcore_barrier()
        pltpu.sync_copy(accA.at[pl.ds(rs, ROWS_PER_SUB)],
                        accA_o.at[c, pl.ds(rs, ROWS_PER_SUB)])

    return sc_attr_scatter


@functools.cache
def _build_sc_geo_scatter():
    @functools.partial(
        pl.kernel,
        out_type=jax.ShapeDtypeStruct((NC, N_PAD, 8), jnp.float32),
        mesh=_mesh(),
        compiler_params=pltpu.CompilerParams(
            use_tc_tiling_on_sc=False, needs_layout_passes=False),
        scratch_types=[
            pltpu.VMEM((2, CHUNK), jnp.int32),
            pltpu.VMEM((2, CHUNK), jnp.int32),
            pltpu.VMEM((2, CHUNK, 8), jnp.float32),
            pltpu.SemaphoreType.DMA,
            pltpu.SemaphoreType.DMA,
            pltpu.SemaphoreType.DMA,
            pltpu.SemaphoreType.DMA,
            pltpu.SemaphoreType.DMA,
            pltpu.SemaphoreType.DMA,
            pltpu.VMEM_SHARED((N_PAD, 8), jnp.float32),
            pltpu.VMEM_SHARED((N_PAD, 8), jnp.float32),
        ],
    )
    def sc_geo_scatter(src_h, dst_h, geo_h, zG_h, accG_o,
                       srcv, dstv, geov, sL0, sL1, sG0, sG1, sS0, sS1, accG,
                       geoT):
        c = lax.axis_index("c")
        s = lax.axis_index("s")
        wid = c * NS + s
        rs = s * ROWS_PER_SUB
        sL = (sL0, sL1)
        sG = (sG0, sG1)
        sS = (sS0, sS1)

        pltpu.sync_copy(zG_h, accG.at[pl.ds(rs, ROWS_PER_SUB)])
        @pl.when(s < NS - 1)
        def _():
            pltpu.sync_copy(geo_h.at[pl.ds(rs, ROWS_PER_SUB)],
                            geoT.at[pl.ds(rs, ROWS_PER_SUB)])

        @pl.when(s == NS - 1)
        def _():
            pltpu.sync_copy(geo_h.at[pl.ds(rs, N - (NS - 1) * ROWS_PER_SUB)],
                            geoT.at[pl.ds(rs, N - (NS - 1) * ROWS_PER_SUB)])
        plsc.subcore_barrier()

        valid, issue_loads, wait_loads, wait_scatters = _pipe_helpers(
            wid, sL, sS, [accG], dstv, [geov],
            [(src_h, srcv), (dst_h, dstv)])

        def step(t, b):
            bn = 1 - b
            wait_scatters(t - 1, bn)
            issue_loads(t + 1, bn)

            @pl.when(valid(t))
            def _():
                wait_loads(t, b)
                pltpu.async_copy(geoT.at[srcv.at[b]], geov.at[b],
                                 sG[b]).wait()
                pltpu.async_copy(geov.at[b], accG.at[dstv.at[b]], sS[b],
                                 add=True)

        issue_loads(0, 0)

        def ebody(j, carry):
            step(2 * j, 0)
            step(2 * j + 1, 1)
            return carry

        lax.fori_loop(0, ETRIPS // 2, ebody, 0)
        wait_scatters(ETRIPS - 1, 1)

        plsc.subcore_barrier()
        pltpu.sync_copy(accG.at[pl.ds(rs, ROWS_PER_SUB)],
                        accG_o.at[c, pl.ds(rs, ROWS_PER_SUB)])

    return sc_geo_scatter


@functools.cache
def _build_sc_gather_masked():
    @functools.partial(
        pl.kernel,
        out_type=(
            jax.ShapeDtypeStruct((NMASK, D0), jnp.float32),
            jax.ShapeDtypeStruct((NMASK, D0), jnp.float32),
        ),
        mesh=_mesh(),
        scratch_types=[
            pltpu.VMEM((MCHUNK,), jnp.int32),
            pltpu.VMEM((MCHUNK, D0), jnp.float32),
            pltpu.VMEM((MCHUNK, D0), jnp.float32),
            pltpu.SemaphoreType.DMA,
            pltpu.SemaphoreType.DMA,
        ],
    )
    def sc_gather_masked(x_h, recon_h, mask_h, xpred_o, xtrue_o,
                         midx, rbuf, xbuf, sem1, sem2):
        c = lax.axis_index("c")
        s = lax.axis_index("s")
        wid = c * NS + s

        def mbody(t, carry):
            mc = wid + NW * t

            @pl.when(mc < NMCHUNKS)
            def _():
                moff = mc * MCHUNK
                pltpu.sync_copy(mask_h.at[pl.ds(moff, MCHUNK)], midx)
                cp1 = pltpu.async_copy(recon_h.at[midx], rbuf, sem1)
                cp2 = pltpu.async_copy(x_h.at[midx], xbuf, sem2)
                cp1.wait()
                pltpu.sync_copy(rbuf, xpred_o.at[pl.ds(moff, MCHUNK)])
                cp2.wait()
                pltpu.sync_copy(xbuf, xtrue_o.at[pl.ds(moff, MCHUNK)])
            return carry

        lax.fori_loop(0, MTRIPS, mbody, 0)

    return sc_gather_masked


def _sc_segment_sum(src, dst, feat, dda, geo, mask32, zA, zG):
    accA, flagbuf = _build_sc_attr_scatter()(dst, feat, dda, mask32, zA)
    accG = _build_sc_geo_scatter()(src, dst, geo, zG)
    return accA, accG, flagbuf


def _sc_gather_masked(x, recon, mask32):
    return _build_sc_gather_masked()(x, recon, mask32)


BN = 1000
GRID = N // BN


def _ln_k(z, g, b, eps=1e-5):
    m = jnp.mean(z, axis=-1, keepdims=True)
    v = jnp.mean((z - m) * (z - m), axis=-1, keepdims=True)
    return (z - m) * lax.rsqrt(v + eps) * g + b


def _tc_body(xb, aA0, aA1, aG0, aG1, flb, normb, tok,
             WxT, WgT, WaT, encb, encg, encbeta,
             npWT, npb, npg, npbeta, e2dWT,
             decWrT, decGT, decAT, decb, decg, decbeta,
             ns_o, rec_o):
    nrm = normb[...]
    ahg = (aG0[0] + aG1[0]) * nrm
    aha = (aA0[0] + aA1[0]) * nrm
    fl = flb[...]
    ox = xb[...] * (1.0 - fl) + tok[...] * fl
    dot = functools.partial(jnp.dot, preferred_element_type=jnp.float32)
    pre = (dot(ox, WxT[...]) + dot(ahg, WgT[...]) + dot(aha, WaT[...])
           + encb[...])
    enc = jnp.maximum(_ln_k(pre, encg[...], encbeta[...]), 0.0)
    ns_o[...] = _ln_k(dot(enc, npWT[...]) + npb[...], npg[...], npbeta[...])
    rep = dot(enc, e2dWT[...])
    dpre = (dot(rep, decWrT[...]) + dot(ahg, decGT[...]) + dot(aha, decAT[...])
            + decb[...])
    rec_o[...] = jnp.maximum(_ln_k(dpre, decg[...], decbeta[...]), 0.0)


def _row_spec(w):
    return pl.BlockSpec((BN, w), lambda i: (i, 0))


def _part_spec(w):
    return (pl.BlockSpec((1, BN, w), lambda i: (0, i, 0)),
            pl.BlockSpec((1, BN, w), lambda i: (1, i, 0)))


def _full_spec(a):
    r = len(a.shape)
    return pl.BlockSpec(a.shape, lambda i: (0,) * r)


def _tc_dense(x, accA, accG, flag, norm, tok, weights):
    aspec0, aspec1 = _part_spec(16)
    gspec0, gspec1 = _part_spec(8)
    wspecs = [_full_spec(w) for w in weights]
    return pl.pallas_call(
        _tc_body,
        grid=(GRID,),
        in_specs=[_row_spec(D0), aspec0, aspec1, gspec0, gspec1,
                  _row_spec(1), _row_spec(1), _full_spec(tok)] + wspecs,
        out_specs=[_row_spec(16), _row_spec(D0)],
        out_shape=[jax.ShapeDtypeStruct((N, 16), jnp.float32),
                   jax.ShapeDtypeStruct((N, D0), jnp.float32)],
    )(x, accA, accA, accG, accG, flag, norm, tok, *weights)


def kernel(x, edge_index, geometric, norm, distance, angle, feat,
           discrete_bin_edges, mask_nodes, enc_mask_token, enc_W, enc_b,
           enc_ln_g, enc_ln_b, e2d_W, np_W, np_b, np_ln_g, np_ln_b, dec_W,
           dec_b, dec_ln_g, dec_ln_b):
    src = edge_index[0].astype(jnp.int32)
    dst = edge_index[1].astype(jnp.int32)
    mask32 = mask_nodes.astype(jnp.int32)
    dda = jnp.concatenate(
        [discrete_bin_edges, distance[:, None], angle[:, None]], axis=1)
    zA = jnp.zeros((ROWS_PER_SUB, 16), jnp.float32)
    zG = jnp.zeros((ROWS_PER_SUB, 8), jnp.float32)
    accA, accG, flagbuf = _sc_segment_sum(src, dst, feat, dda, geometric,
                                          mask32, zA, zG)
    flag = flagbuf.reshape(N_PAD, 1)

    weights = (
        enc_W[:, :D0].T,
        enc_W[:, D0:D0 + 8].T,
        enc_W[:, _PERM_E].T,
        enc_b[None, :], enc_ln_g[None, :], enc_ln_b[None, :],
        np_W.T, np_b[None, :], np_ln_g[None, :], np_ln_b[None, :],
        e2d_W.T,
        dec_W[:, :D1].T,
        dec_W[:, D1:D1 + 8].T,
        dec_W[:, _PERM_D].T,
        dec_b[None, :], dec_ln_g[None, :], dec_ln_b[None, :],
    )
    n_scores, recon = _tc_dense(x, accA, accG, flag, norm, enc_mask_token,
                                weights)
    x_pred, x_true = _sc_gather_masked(x, recon, mask32)
    return (x_pred, x_true, n_scores)


# speedup vs baseline: 1.6532x; 1.6532x over previous
import functools

import jax
import jax.numpy as jnp
from jax import lax
from jax.experimental import pallas as pl
from jax.experimental.pallas import tpu as pltpu
from jax.experimental.pallas import tpu_sc as plsc

N = 50000
E = 800000
D0 = 128
D1 = 64
NMASK = 10000
FEAT_W = 8
DISC_W = 6

NC = 2
NS = 16
NW = NC * NS

CHUNK = 128
NCHUNKS = E // CHUNK
ETRIPS = (NCHUNKS + NW - 1) // NW
ROWS_PER_SUB = 3128
N_PAD = NS * ROWS_PER_SUB
MCHUNK = 16
PKR = CHUNK // 8
NMCHUNKS = NMASK // MCHUNK
MTRIPS = (NMCHUNKS + NW - 1) // NW

def _mesh():
    return plsc.VectorSubcoreMesh(
        core_axis_name="c", subcore_axis_name="s",
        num_cores=NC, num_subcores=NS)


FROWS = N_PAD // 16

_PERM_E = ([D0 + 10 + j for j in range(8)] + [D0 + 18 + j for j in range(6)]
           + [D0 + 8, D0 + 9])
_PERM_D = ([D1 + 10 + j for j in range(8)] + [D1 + 18 + j for j in range(6)]
           + [D1 + 8, D1 + 9])


def _pipe_helpers(wid, sL, sS, acc, dstv, datv, loads):
    def _valid(t):
        return (t >= 0) & ((wid + NW * t) < NCHUNKS)

    def _off(t):
        return (wid + NW * t) * CHUNK

    def _issue_loads(t, b):
        @pl.when(_valid(t))
        def _():
            off = _off(t)
            for h, v in loads:
                pltpu.async_copy(h.at[pl.ds(off, CHUNK)], v.at[b], sL[b])

    def _wait_loads(t, b):
        @pl.when(_valid(t))
        def _():
            off = _off(t)
            for h, v in loads:
                pltpu.make_async_copy(h.at[pl.ds(off, CHUNK)], v.at[b],
                                      sL[b]).wait()

    def _wait_scatters(t, b):
        @pl.when(_valid(t))
        def _():
            for a, v in zip(acc, datv):
                pltpu.make_async_copy(v.at[b], a.at[dstv.at[b]], sS[b]).wait()

    return _valid, _issue_loads, _wait_loads, _wait_scatters


@functools.cache
def _build_sc_attr_scatter():
    @functools.partial(
        pl.kernel,
        out_type=(
            jax.ShapeDtypeStruct((NC, N_PAD, 16), jnp.float32),
            jax.ShapeDtypeStruct((FROWS, 16), jnp.float32),
        ),
        mesh=_mesh(),
        compiler_params=pltpu.CompilerParams(
            use_tc_tiling_on_sc=False, needs_layout_passes=False),
        scratch_types=[
            pltpu.VMEM((2, CHUNK), jnp.int32),
            pltpu.VMEM((2, CHUNK, 16), jnp.float32),
            pltpu.VMEM((NMASK,), jnp.int32),
            pltpu.VMEM((ROWS_PER_SUB, 16), jnp.float32),
            pltpu.SemaphoreType.DMA,
            pltpu.SemaphoreType.DMA,
            pltpu.SemaphoreType.DMA,
            pltpu.SemaphoreType.DMA,
            pltpu.VMEM_SHARED((N_PAD, 16), jnp.float32),
        ],
    )
    def sc_attr_scatter(dst_h, attr_h, mask_h, zA_h, accA_o, flag_o,
                        dstv, attrv, midxv, zb16, sL0, sL1, sS0, sS1, accA):
        c = lax.axis_index("c")
        s = lax.axis_index("s")
        wid = c * NS + s
        rs = s * ROWS_PER_SUB
        sL = (sL0, sL1)
        sS = (sS0, sS1)

        pltpu.sync_copy(zA_h, accA.at[pl.ds(rs, ROWS_PER_SUB)])

        @pl.when(wid == 0)
        def _():
            pltpu.sync_copy(zA_h, zb16)
            pltpu.sync_copy(mask_h, midxv)
            ones = jnp.full((MCHUNK,), 1.0, jnp.float32)

            def fbody(t, carry):
                mi = midxv[pl.ds(t * MCHUNK, MCHUNK)]
                plsc.store_scatter(zb16, [mi >> 4, mi & 15], ones)
                return carry

            lax.fori_loop(0, NMCHUNKS, fbody, 0)
            pltpu.sync_copy(zb16.at[pl.ds(0, FROWS)], flag_o)

        plsc.subcore_barrier()

        def valid(t):
            return (t >= 0) & ((wid + NW * t) < NCHUNKS)

        def issue_loads(t, b):
            @pl.when(valid(t))
            def _():
                ci = wid + NW * t
                pltpu.async_copy(dst_h.at[pl.ds(ci * CHUNK, CHUNK)],
                                 dstv.at[b], sL[b])
                pltpu.async_copy(attr_h.at[pl.ds(ci * CHUNK, CHUNK)],
                                 attrv.at[b], sL[b])

        def wait_loads(t, b):
            @pl.when(valid(t))
            def _():
                ci = wid + NW * t
                pltpu.make_async_copy(dst_h.at[pl.ds(ci * CHUNK, CHUNK)],
                                      dstv.at[b], sL[b]).wait()
                pltpu.make_async_copy(attr_h.at[pl.ds(ci * CHUNK, CHUNK)],
                                      attrv.at[b], sL[b]).wait()

        def wait_scatters(t, b):
            @pl.when(valid(t))
            def _():
                pltpu.make_async_copy(attrv.at[b], accA.at[dstv.at[b]],
                                      sS[b]).wait()

        def step(t, b):
            bn = 1 - b
            wait_scatters(t - 1, bn)
            issue_loads(t + 1, bn)

            @pl.when(valid(t))
            def _():
                wait_loads(t, b)
                pltpu.async_copy(attrv.at[b], accA.at[dstv.at[b]], sS[b],
                                 add=True)

        issue_loads(0, 0)

        def ebody(j, carry):
            step(2 * j, 0)
            step(2 * j + 1, 1)
            return carry

        lax.fori_loop(0, ETRIPS // 2, ebody, 0)
        wait_scatters(ETRIPS - 1, 1)

        plsc.subcore_barrier()
        pltpu.sync_copy(accA.at[pl.ds(rs, ROWS_PER_SUB)],
                        accA_o.at[c, pl.ds(rs, ROWS_PER_SUB)])

    return sc_attr_scatter


@functools.cache
def _build_sc_geo_scatter():
    @functools.partial(
        pl.kernel,
        out_type=jax.ShapeDtypeStruct((NC, N_PAD, 8), jnp.float32),
        mesh=_mesh(),
        compiler_params=pltpu.CompilerParams(
            use_tc_tiling_on_sc=False, needs_layout_passes=False),
        scratch_types=[
            pltpu.VMEM((2, CHUNK), jnp.int32),
            pltpu.VMEM((2, CHUNK), jnp.int32),
            pltpu.VMEM((2, CHUNK, 8), jnp.float32),
            pltpu.SemaphoreType.DMA,
            pltpu.SemaphoreType.DMA,
            pltpu.SemaphoreType.DMA,
            pltpu.SemaphoreType.DMA,
            pltpu.SemaphoreType.DMA,
            pltpu.SemaphoreType.DMA,
            pltpu.VMEM_SHARED((N_PAD, 8), jnp.float32),
            pltpu.VMEM_SHARED((N_PAD, 8), jnp.float32),
        ],
    )
    def sc_geo_scatter(src_h, dst_h, geo_h, zG_h, accG_o,
                       srcv, dstv, geov, sL0, sL1, sG0, sG1, sS0, sS1, accG,
                       geoT):
        c = lax.axis_index("c")
        s = lax.axis_index("s")
        wid = c * NS + s
        rs = s * ROWS_PER_SUB
        sL = (sL0, sL1)
        sG = (sG0, sG1)
        sS = (sS0, sS1)

        pltpu.sync_copy(zG_h, accG.at[pl.ds(rs, ROWS_PER_SUB)])
        @pl.when(s < NS - 1)
        def _():
            pltpu.sync_copy(geo_h.at[pl.ds(rs, ROWS_PER_SUB)],
                            geoT.at[pl.ds(rs, ROWS_PER_SUB)])

        @pl.when(s == NS - 1)
        def _():
            pltpu.sync_copy(geo_h.at[pl.ds(rs, N - (NS - 1) * ROWS_PER_SUB)],
                            geoT.at[pl.ds(rs, N - (NS - 1) * ROWS_PER_SUB)])
        plsc.subcore_barrier()

        valid, issue_loads, wait_loads, wait_scatters = _pipe_helpers(
            wid, sL, sS, [accG], dstv, [geov],
            [(src_h, srcv), (dst_h, dstv)])

        def step(t, b):
            bn = 1 - b
            wait_scatters(t - 1, bn)
            issue_loads(t + 1, bn)

            @pl.when(valid(t))
            def _():
                wait_loads(t, b)
                pltpu.async_copy(geoT.at[srcv.at[b]], geov.at[b],
                                 sG[b]).wait()
                pltpu.async_copy(geov.at[b], accG.at[dstv.at[b]], sS[b],
                                 add=True)

        issue_loads(0, 0)

        def ebody(j, carry):
            step(2 * j, 0)
            step(2 * j + 1, 1)
            return carry

        lax.fori_loop(0, ETRIPS // 2, ebody, 0)
        wait_scatters(ETRIPS - 1, 1)

        plsc.subcore_barrier()
        pltpu.sync_copy(accG.at[pl.ds(rs, ROWS_PER_SUB)],
                        accG_o.at[c, pl.ds(rs, ROWS_PER_SUB)])

    return sc_geo_scatter


@functools.cache
def _build_sc_gather_masked():
    @functools.partial(
        pl.kernel,
        out_type=(
            jax.ShapeDtypeStruct((NMASK, D0), jnp.float32),
            jax.ShapeDtypeStruct((NMASK, D0), jnp.float32),
        ),
        mesh=_mesh(),
        scratch_types=[
            pltpu.VMEM((MCHUNK,), jnp.int32),
            pltpu.VMEM((MCHUNK, D0), jnp.float32),
            pltpu.VMEM((MCHUNK, D0), jnp.float32),
            pltpu.SemaphoreType.DMA,
            pltpu.SemaphoreType.DMA,
        ],
    )
    def sc_gather_masked(x_h, recon_h, mask_h, xpred_o, xtrue_o,
                         midx, rbuf, xbuf, sem1, sem2):
        c = lax.axis_index("c")
        s = lax.axis_index("s")
        wid = c * NS + s

        def mbody(t, carry):
            mc = wid + NW * t

            @pl.when(mc < NMCHUNKS)
            def _():
                moff = mc * MCHUNK
                pltpu.sync_copy(mask_h.at[pl.ds(moff, MCHUNK)], midx)
                cp1 = pltpu.async_copy(recon_h.at[midx], rbuf, sem1)
                cp2 = pltpu.async_copy(x_h.at[midx], xbuf, sem2)
                cp1.wait()
                pltpu.sync_copy(rbuf, xpred_o.at[pl.ds(moff, MCHUNK)])
                cp2.wait()
                pltpu.sync_copy(xbuf, xtrue_o.at[pl.ds(moff, MCHUNK)])
            return carry

        lax.fori_loop(0, MTRIPS, mbody, 0)

    return sc_gather_masked


def _sc_segment_sum(src, dst, attr, geo, mask32, zA, zG):
    accA, flagbuf = _build_sc_attr_scatter()(dst, attr, mask32, zA)
    accG = _build_sc_geo_scatter()(src, dst, geo, zG)
    return accA, accG, flagbuf


def _sc_gather_masked(x, recon, mask32):
    return _build_sc_gather_masked()(x, recon, mask32)


BN = 1000
GRID = N // BN


def _ln_k(z, g, b, eps=1e-5):
    m = jnp.mean(z, axis=-1, keepdims=True)
    v = jnp.mean((z - m) * (z - m), axis=-1, keepdims=True)
    return (z - m) * lax.rsqrt(v + eps) * g + b


def _tc_body(xb, aA0, aA1, aG0, aG1, flb, normb, tok,
             WxT, WgT, WaT, encb, encg, encbeta,
             npWT, npb, npg, npbeta, e2dWT,
             decWrT, decGT, decAT, decb, decg, decbeta,
             ns_o, rec_o):
    nrm = normb[...]
    ahg = (aG0[0] + aG1[0]) * nrm
    aha = (aA0[0] + aA1[0]) * nrm
    fl = flb[...]
    ox = xb[...] * (1.0 - fl) + tok[...] * fl
    dot = functools.partial(jnp.dot, preferred_element_type=jnp.float32)
    pre = (dot(ox, WxT[...]) + dot(ahg, WgT[...]) + dot(aha, WaT[...])
           + encb[...])
    enc = jnp.maximum(_ln_k(pre, encg[...], encbeta[...]), 0.0)
    ns_o[...] = _ln_k(dot(enc, npWT[...]) + npb[...], npg[...], npbeta[...])
    rep = dot(enc, e2dWT[...])
    dpre = (dot(rep, decWrT[...]) + dot(ahg, decGT[...]) + dot(aha, decAT[...])
            + decb[...])
    rec_o[...] = jnp.maximum(_ln_k(dpre, decg[...], decbeta[...]), 0.0)


def _row_spec(w):
    return pl.BlockSpec((BN, w), lambda i: (i, 0))


def _part_spec(w):
    return (pl.BlockSpec((1, BN, w), lambda i: (0, i, 0)),
            pl.BlockSpec((1, BN, w), lambda i: (1, i, 0)))


def _full_spec(a):
    r = len(a.shape)
    return pl.BlockSpec(a.shape, lambda i: (0,) * r)


def _tc_dense(x, accA, accG, flag, norm, tok, weights):
    aspec0, aspec1 = _part_spec(16)
    gspec0, gspec1 = _part_spec(8)
    wspecs = [_full_spec(w) for w in weights]
    return pl.pallas_call(
        _tc_body,
        grid=(GRID,),
        in_specs=[_row_spec(D0), aspec0, aspec1, gspec0, gspec1,
                  _row_spec(1), _row_spec(1), _full_spec(tok)] + wspecs,
        out_specs=[_row_spec(16), _row_spec(D0)],
        out_shape=[jax.ShapeDtypeStruct((N, 16), jnp.float32),
                   jax.ShapeDtypeStruct((N, D0), jnp.float32)],
    )(x, accA, accA, accG, accG, flag, norm, tok, *weights)


def kernel(x, edge_index, geometric, norm, distance, angle, feat,
           discrete_bin_edges, mask_nodes, enc_mask_token, enc_W, enc_b,
           enc_ln_g, enc_ln_b, e2d_W, np_W, np_b, np_ln_g, np_ln_b, dec_W,
           dec_b, dec_ln_g, dec_ln_b):
    src = edge_index[0].astype(jnp.int32)
    dst = edge_index[1].astype(jnp.int32)
    mask32 = mask_nodes.astype(jnp.int32)
    attr = jnp.concatenate(
        [distance[:, None], angle[:, None], feat, discrete_bin_edges], axis=1)
    zA = jnp.zeros((ROWS_PER_SUB, 16), jnp.float32)
    zG = jnp.zeros((ROWS_PER_SUB, 8), jnp.float32)
    accA, accG, flagbuf = _sc_segment_sum(src, dst, attr, geometric,
                                          mask32, zA, zG)
    flag = flagbuf.reshape(N_PAD, 1)

    weights = (
        enc_W[:, :D0].T,
        enc_W[:, D0:D0 + 8].T,
        enc_W[:, D0 + 8:D0 + 24].T,
        enc_b[None, :], enc_ln_g[None, :], enc_ln_b[None, :],
        np_W.T, np_b[None, :], np_ln_g[None, :], np_ln_b[None, :],
        e2d_W.T,
        dec_W[:, :D1].T,
        dec_W[:, D1:D1 + 8].T,
        dec_W[:, D1 + 8:D1 + 24].T,
        dec_b[None, :], dec_ln_g[None, :], dec_ln_b[None, :],
    )
    n_scores, recon = _tc_dense(x, accA, accG, flag, norm, enc_mask_token,
                                weights)
    x_pred, x_true = _sc_gather_masked(x, recon, mask32)
    return (x_pred, x_true, n_scores)


# speedup vs baseline: 1.7131x; 1.0362x over previous
import functools

import jax
import jax.numpy as jnp
from jax import lax
from jax.experimental import pallas as pl
from jax.experimental.pallas import tpu as pltpu
from jax.experimental.pallas import tpu_sc as plsc

N = 50000
E = 800000
D0 = 128
D1 = 64
NMASK = 10000
FEAT_W = 8
DISC_W = 6

NC = 2
NS = 16
NW = NC * NS

CHUNK = 128
NCHUNKS = E // CHUNK
ETRIPS = (NCHUNKS + NW - 1) // NW
ROWS_PER_SUB = 3128
N_PAD = NS * ROWS_PER_SUB
MCHUNK = 16
PKR = CHUNK // 8
NMCHUNKS = NMASK // MCHUNK
MTRIPS = (NMCHUNKS + NW - 1) // NW

def _mesh():
    return plsc.VectorSubcoreMesh(
        core_axis_name="c", subcore_axis_name="s",
        num_cores=NC, num_subcores=NS)


FROWS = N_PAD // 16

_PERM_E = ([D0 + 10 + j for j in range(8)] + [D0 + 18 + j for j in range(6)]
           + [D0 + 8, D0 + 9])
_PERM_D = ([D1 + 10 + j for j in range(8)] + [D1 + 18 + j for j in range(6)]
           + [D1 + 8, D1 + 9])


def _pipe_helpers(wid, sL, sS, acc, dstv, datv, loads):
    def _valid(t):
        return (t >= 0) & ((wid + NW * t) < NCHUNKS)

    def _off(t):
        return (wid + NW * t) * CHUNK

    def _issue_loads(t, b):
        @pl.when(_valid(t))
        def _():
            off = _off(t)
            for h, v in loads:
                pltpu.async_copy(h.at[pl.ds(off, CHUNK)], v.at[b], sL[b])

    def _wait_loads(t, b):
        @pl.when(_valid(t))
        def _():
            off = _off(t)
            for h, v in loads:
                pltpu.make_async_copy(h.at[pl.ds(off, CHUNK)], v.at[b],
                                      sL[b]).wait()

    def _wait_scatters(t, b):
        @pl.when(_valid(t))
        def _():
            for a, v in zip(acc, datv):
                pltpu.make_async_copy(v.at[b], a.at[dstv.at[b]], sS[b]).wait()

    return _valid, _issue_loads, _wait_loads, _wait_scatters


@functools.cache
def _build_sc_attr_scatter():
    @functools.partial(
        pl.kernel,
        out_type=(
            jax.ShapeDtypeStruct((NC, N_PAD, 16), jnp.float32),
            jax.ShapeDtypeStruct((FROWS, 16), jnp.float32),
        ),
        mesh=_mesh(),
        compiler_params=pltpu.CompilerParams(
            use_tc_tiling_on_sc=False, needs_layout_passes=False),
        scratch_types=[
            pltpu.VMEM((2, CHUNK), jnp.int32),
            pltpu.VMEM((2, CHUNK, 16), jnp.float32),
            pltpu.VMEM((NMASK,), jnp.int32),
            pltpu.VMEM((ROWS_PER_SUB, 16), jnp.float32),
            pltpu.SemaphoreType.DMA,
            pltpu.SemaphoreType.DMA,
            pltpu.SemaphoreType.DMA,
            pltpu.SemaphoreType.DMA,
            pltpu.VMEM_SHARED((N_PAD, 16), jnp.float32),
        ],
    )
    def sc_attr_scatter(dst_h, attr_h, mask_h, zA_h, accA_o, flag_o,
                        dstv, attrv, midxv, zb16, sL0, sL1, sS0, sS1, accA):
        c = lax.axis_index("c")
        s = lax.axis_index("s")
        wid = c * NS + s
        rs = s * ROWS_PER_SUB
        sL = (sL0, sL1)
        sS = (sS0, sS1)

        pltpu.sync_copy(zA_h, accA.at[pl.ds(rs, ROWS_PER_SUB)])

        @pl.when(wid == 0)
        def _():
            pltpu.sync_copy(zA_h, zb16)
            pltpu.sync_copy(mask_h, midxv)
            ones = jnp.full((MCHUNK,), 1.0, jnp.float32)

            def fbody(t, carry):
                mi = midxv[pl.ds(t * MCHUNK, MCHUNK)]
                plsc.store_scatter(zb16, [mi >> 4, mi & 15], ones)
                return carry

            lax.fori_loop(0, NMCHUNKS, fbody, 0)
            pltpu.sync_copy(zb16.at[pl.ds(0, FROWS)], flag_o)

        plsc.subcore_barrier()

        def valid(t):
            return (t >= 0) & ((wid + NW * t) < NCHUNKS)

        def issue_loads(t, b):
            @pl.when(valid(t))
            def _():
                ci = wid + NW * t
                pltpu.async_copy(dst_h.at[pl.ds(ci * CHUNK, CHUNK)],
                                 dstv.at[b], sL[b])
                pltpu.async_copy(attr_h.at[pl.ds(ci * CHUNK, CHUNK)],
                                 attrv.at[b], sL[b])

        def wait_loads(t, b):
            @pl.when(valid(t))
            def _():
                ci = wid + NW * t
                pltpu.make_async_copy(dst_h.at[pl.ds(ci * CHUNK, CHUNK)],
                                      dstv.at[b], sL[b]).wait()
                pltpu.make_async_copy(attr_h.at[pl.ds(ci * CHUNK, CHUNK)],
                                      attrv.at[b], sL[b]).wait()

        def wait_scatters(t, b):
            @pl.when(valid(t))
            def _():
                pltpu.make_async_copy(attrv.at[b], accA.at[dstv.at[b]],
                                      sS[b]).wait()

        def step(t, b):
            bn = 1 - b
            wait_scatters(t - 1, bn)
            issue_loads(t + 1, bn)

            @pl.when(valid(t))
            def _():
                wait_loads(t, b)
                pltpu.async_copy(attrv.at[b], accA.at[dstv.at[b]], sS[b],
                                 add=True)

        issue_loads(0, 0)

        def ebody(j, carry):
            step(2 * j, 0)
            step(2 * j + 1, 1)
            return carry

        lax.fori_loop(0, ETRIPS // 2, ebody, 0)
        wait_scatters(ETRIPS - 1, 1)

        plsc.subcore_barrier()
        pltpu.sync_copy(accA.at[pl.ds(rs, ROWS_PER_SUB)],
                        accA_o.at[c, pl.ds(rs, ROWS_PER_SUB)])

    return sc_attr_scatter


@functools.cache
def _build_sc_geo_scatter():
    @functools.partial(
        pl.kernel,
        out_type=jax.ShapeDtypeStruct((NC, N_PAD, 8), jnp.float32),
        mesh=_mesh(),
        compiler_params=pltpu.CompilerParams(
            use_tc_tiling_on_sc=False, needs_layout_passes=False),
        scratch_types=[
            pltpu.VMEM((2, CHUNK), jnp.int32),
            pltpu.VMEM((2, CHUNK), jnp.int32),
            pltpu.VMEM((2, CHUNK, 8), jnp.float32),
            pltpu.SemaphoreType.DMA,
            pltpu.SemaphoreType.DMA,
            pltpu.SemaphoreType.DMA,
            pltpu.SemaphoreType.DMA,
            pltpu.SemaphoreType.DMA,
            pltpu.SemaphoreType.DMA,
            pltpu.VMEM_SHARED((N_PAD, 8), jnp.float32),
            pltpu.VMEM_SHARED((N_PAD, 8), jnp.float32),
        ],
    )
    def sc_geo_scatter(src_h, dst_h, geo_h, zG_h, accG_o,
                       srcv, dstv, geov, sL0, sL1, sG0, sG1, sS0, sS1, accG,
                       geoT):
        c = lax.axis_index("c")
        s = lax.axis_index("s")
        wid = c * NS + s
        rs = s * ROWS_PER_SUB
        sL = (sL0, sL1)
        sG = (sG0, sG1)
        sS = (sS0, sS1)

        pltpu.sync_copy(zG_h, accG.at[pl.ds(rs, ROWS_PER_SUB)])
        @pl.when(s < NS - 1)
        def _():
            pltpu.sync_copy(geo_h.at[pl.ds(rs, ROWS_PER_SUB)],
                            geoT.at[pl.ds(rs, ROWS_PER_SUB)])

        @pl.when(s == NS - 1)
        def _():
            pltpu.sync_copy(geo_h.at[pl.ds(rs, N - (NS - 1) * ROWS_PER_SUB)],
                            geoT.at[pl.ds(rs, N - (NS - 1) * ROWS_PER_SUB)])
        plsc.subcore_barrier()

        valid, issue_loads, wait_loads, wait_scatters = _pipe_helpers(
            wid, sL, sS, [accG], dstv, [geov],
            [(src_h, srcv), (dst_h, dstv)])

        def step(t, b):
            bn = 1 - b
            wait_scatters(t - 1, bn)
            issue_loads(t + 1, bn)

            @pl.when(valid(t))
            def _():
                wait_loads(t, b)
                pltpu.async_copy(geoT.at[srcv.at[b]], geov.at[b],
                                 sG[b]).wait()
                pltpu.async_copy(geov.at[b], accG.at[dstv.at[b]], sS[b],
                                 add=True)

        issue_loads(0, 0)

        def ebody(j, carry):
            step(2 * j, 0)
            step(2 * j + 1, 1)
            return carry

        lax.fori_loop(0, ETRIPS // 2, ebody, 0)
        wait_scatters(ETRIPS - 1, 1)

        plsc.subcore_barrier()
        pltpu.sync_copy(accG.at[pl.ds(rs, ROWS_PER_SUB)],
                        accG_o.at[c, pl.ds(rs, ROWS_PER_SUB)])

    return sc_geo_scatter


@functools.cache
def _build_sc_gather_masked():
    @functools.partial(
        pl.kernel,
        out_type=(
            jax.ShapeDtypeStruct((NMASK, D0), jnp.float32),
            jax.ShapeDtypeStruct((NMASK, D0), jnp.float32),
        ),
        mesh=_mesh(),
        scratch_types=[
            pltpu.VMEM((MCHUNK,), jnp.int32),
            pltpu.VMEM((MCHUNK, D0), jnp.float32),
            pltpu.VMEM((MCHUNK, D0), jnp.float32),
            pltpu.SemaphoreType.DMA,
            pltpu.SemaphoreType.DMA,
        ],
    )
    def sc_gather_masked(x_h, recon_h, mask_h, xpred_o, xtrue_o,
                         midx, rbuf, xbuf, sem1, sem2):
        c = lax.axis_index("c")
        s = lax.axis_index("s")
        wid = c * NS + s

        def mbody(t, carry):
            mc = wid + NW * t

            @pl.when(mc < NMCHUNKS)
            def _():
                moff = mc * MCHUNK
                pltpu.sync_copy(mask_h.at[pl.ds(moff, MCHUNK)], midx)
                cp1 = pltpu.async_copy(recon_h.at[midx], rbuf, sem1)
                cp2 = pltpu.async_copy(x_h.at[midx], xbuf, sem2)
                cp1.wait()
                pltpu.sync_copy(rbuf, xpred_o.at[pl.ds(moff, MCHUNK)])
                cp2.wait()
                pltpu.sync_copy(xbuf, xtrue_o.at[pl.ds(moff, MCHUNK)])
            return carry

        lax.fori_loop(0, MTRIPS, mbody, 0)

    return sc_gather_masked


def _sc_segment_sum(src, dst, attr, geo, mask32, zA, zG):
    accA, flagbuf = _build_sc_attr_scatter()(dst, attr, mask32, zA)
    accG = _build_sc_geo_scatter()(src, dst, geo, zG)
    return accA, accG, flagbuf


def _sc_gather_masked(x, recon, mask32):
    return _build_sc_gather_masked()(x, recon, mask32)


BN = 2000
GRID = N // BN


def _ln_k(z, g, b, eps=1e-5):
    m = jnp.mean(z, axis=-1, keepdims=True)
    v = jnp.mean((z - m) * (z - m), axis=-1, keepdims=True)
    return (z - m) * lax.rsqrt(v + eps) * g + b


def _tc_body(xb, aA0, aA1, aG0, aG1, flb, normb, tok,
             WxT, WgT, WaT, encb, encg, encbeta,
             npWT, npb, npg, npbeta, e2dWT,
             decWrT, decGT, decAT, decb, decg, decbeta,
             ns_o, rec_o):
    nrm = normb[...]
    ahg = (aG0[0] + aG1[0]) * nrm
    aha = (aA0[0] + aA1[0]) * nrm
    fl = flb[...]
    ox = xb[...] * (1.0 - fl) + tok[...] * fl
    dot = functools.partial(jnp.dot, preferred_element_type=jnp.float32)
    pre = (dot(ox, WxT[...]) + dot(ahg, WgT[...]) + dot(aha, WaT[...])
           + encb[...])
    enc = jnp.maximum(_ln_k(pre, encg[...], encbeta[...]), 0.0)
    ns_o[...] = _ln_k(dot(enc, npWT[...]) + npb[...], npg[...], npbeta[...])
    rep = dot(enc, e2dWT[...])
    dpre = (dot(rep, decWrT[...]) + dot(ahg, decGT[...]) + dot(aha, decAT[...])
            + decb[...])
    rec_o[...] = jnp.maximum(_ln_k(dpre, decg[...], decbeta[...]), 0.0)


def _row_spec(w):
    return pl.BlockSpec((BN, w), lambda i: (i, 0))


def _part_spec(w):
    return (pl.BlockSpec((1, BN, w), lambda i: (0, i, 0)),
            pl.BlockSpec((1, BN, w), lambda i: (1, i, 0)))


def _full_spec(a):
    r = len(a.shape)
    return pl.BlockSpec(a.shape, lambda i: (0,) * r)


def _tc_dense(x, accA, accG, flag, norm, tok, weights):
    aspec0, aspec1 = _part_spec(16)
    gspec0, gspec1 = _part_spec(8)
    wspecs = [_full_spec(w) for w in weights]
    return pl.pallas_call(
        _tc_body,
        grid=(GRID,),
        in_specs=[_row_spec(D0), aspec0, aspec1, gspec0, gspec1,
                  _row_spec(1), _row_spec(1), _full_spec(tok)] + wspecs,
        out_specs=[_row_spec(16), _row_spec(D0)],
        out_shape=[jax.ShapeDtypeStruct((N, 16), jnp.float32),
                   jax.ShapeDtypeStruct((N, D0), jnp.float32)],
    )(x, accA, accA, accG, accG, flag, norm, tok, *weights)


def kernel(x, edge_index, geometric, norm, distance, angle, feat,
           discrete_bin_edges, mask_nodes, enc_mask_token, enc_W, enc_b,
           enc_ln_g, enc_ln_b, e2d_W, np_W, np_b, np_ln_g, np_ln_b, dec_W,
           dec_b, dec_ln_g, dec_ln_b):
    src = edge_index[0].astype(jnp.int32)
    dst = edge_index[1].astype(jnp.int32)
    mask32 = mask_nodes.astype(jnp.int32)
    attr = jnp.concatenate(
        [distance[:, None], angle[:, None], feat, discrete_bin_edges], axis=1)
    zA = jnp.zeros((ROWS_PER_SUB, 16), jnp.float32)
    zG = jnp.zeros((ROWS_PER_SUB, 8), jnp.float32)
    accA, accG, flagbuf = _sc_segment_sum(src, dst, attr, geometric,
                                          mask32, zA, zG)
    flag = flagbuf.reshape(N_PAD, 1)

    weights = (
        enc_W[:, :D0].T,
        enc_W[:, D0:D0 + 8].T,
        enc_W[:, D0 + 8:D0 + 24].T,
        enc_b[None, :], enc_ln_g[None, :], enc_ln_b[None, :],
        np_W.T, np_b[None, :], np_ln_g[None, :], np_ln_b[None, :],
        e2d_W.T,
        dec_W[:, :D1].T,
        dec_W[:, D1:D1 + 8].T,
        dec_W[:, D1 + 8:D1 + 24].T,
        dec_b[None, :], dec_ln_g[None, :], dec_ln_b[None, :],
    )
    n_scores, recon = _tc_dense(x, accA, accG, flag, norm, enc_mask_token,
                                weights)
    x_pred, x_true = _sc_gather_masked(x, recon, mask32)
    return (x_pred, x_true, n_scores)


# speedup vs baseline: 1.7182x; 1.0030x over previous
import functools

import jax
import jax.numpy as jnp
from jax import lax
from jax.experimental import pallas as pl
from jax.experimental.pallas import tpu as pltpu
from jax.experimental.pallas import tpu_sc as plsc

N = 50000
E = 800000
D0 = 128
D1 = 64
NMASK = 10000
FEAT_W = 8
DISC_W = 6

NC = 2
NS = 16
NW = NC * NS

CHUNK = 128
NCHUNKS = E // CHUNK
ETRIPS = (NCHUNKS + NW - 1) // NW
ROWS_PER_SUB = 3128
N_PAD = NS * ROWS_PER_SUB
MCHUNK = 16
PKR = CHUNK // 8
NMCHUNKS = NMASK // MCHUNK
MTRIPS = (NMCHUNKS + NW - 1) // NW

def _mesh():
    return plsc.VectorSubcoreMesh(
        core_axis_name="c", subcore_axis_name="s",
        num_cores=NC, num_subcores=NS)


FROWS = N_PAD // 16

_PERM_E = ([D0 + 10 + j for j in range(8)] + [D0 + 18 + j for j in range(6)]
           + [D0 + 8, D0 + 9])
_PERM_D = ([D1 + 10 + j for j in range(8)] + [D1 + 18 + j for j in range(6)]
           + [D1 + 8, D1 + 9])


def _pipe_helpers(wid, sL, sS, acc, dstv, datv, loads):
    def _valid(t):
        return (t >= 0) & ((wid + NW * t) < NCHUNKS)

    def _off(t):
        return (wid + NW * t) * CHUNK

    def _issue_loads(t, b):
        @pl.when(_valid(t))
        def _():
            off = _off(t)
            for h, v in loads:
                pltpu.async_copy(h.at[pl.ds(off, CHUNK)], v.at[b], sL[b])

    def _wait_loads(t, b):
        @pl.when(_valid(t))
        def _():
            off = _off(t)
            for h, v in loads:
                pltpu.make_async_copy(h.at[pl.ds(off, CHUNK)], v.at[b],
                                      sL[b]).wait()

    def _wait_scatters(t, b):
        @pl.when(_valid(t))
        def _():
            for a, v in zip(acc, datv):
                pltpu.make_async_copy(v.at[b], a.at[dstv.at[b]], sS[b]).wait()

    return _valid, _issue_loads, _wait_loads, _wait_scatters


@functools.cache
def _build_sc_attr_scatter():
    @functools.partial(
        pl.kernel,
        out_type=(
            jax.ShapeDtypeStruct((NC, N_PAD, 16), jnp.float32),
            jax.ShapeDtypeStruct((FROWS, 16), jnp.float32),
        ),
        mesh=_mesh(),
        compiler_params=pltpu.CompilerParams(
            use_tc_tiling_on_sc=False, needs_layout_passes=False),
        scratch_types=[
            pltpu.VMEM((2, CHUNK), jnp.int32),
            pltpu.VMEM((2, CHUNK, 16), jnp.float32),
            pltpu.VMEM((NMASK,), jnp.int32),
            pltpu.VMEM((ROWS_PER_SUB, 16), jnp.float32),
            pltpu.SemaphoreType.DMA,
            pltpu.SemaphoreType.DMA,
            pltpu.SemaphoreType.DMA,
            pltpu.SemaphoreType.DMA,
            pltpu.VMEM_SHARED((N_PAD, 16), jnp.float32),
        ],
    )
    def sc_attr_scatter(dst_h, attr_h, mask_h, zA_h, accA_o, flag_o,
                        dstv, attrv, midxv, zb16, sL0, sL1, sS0, sS1, accA):
        c = lax.axis_index("c")
        s = lax.axis_index("s")
        wid = c * NS + s
        rs = s * ROWS_PER_SUB
        sL = (sL0, sL1)
        sS = (sS0, sS1)

        pltpu.sync_copy(zA_h, accA.at[pl.ds(rs, ROWS_PER_SUB)])

        @pl.when(wid == 0)
        def _():
            pltpu.sync_copy(zA_h, zb16)
            pltpu.sync_copy(mask_h, midxv)
            ones = jnp.full((MCHUNK,), 1.0, jnp.float32)

            def fbody(t, carry):
                mi = midxv[pl.ds(t * MCHUNK, MCHUNK)]
                plsc.store_scatter(zb16, [mi >> 4, mi & 15], ones)
                return carry

            lax.fori_loop(0, NMCHUNKS, fbody, 0)
            pltpu.sync_copy(zb16.at[pl.ds(0, FROWS)], flag_o)

        plsc.subcore_barrier()

        def valid(t):
            return (t >= 0) & ((wid + NW * t) < NCHUNKS)

        def issue_loads(t, b):
            @pl.when(valid(t))
            def _():
                ci = wid + NW * t
                pltpu.async_copy(dst_h.at[pl.ds(ci * CHUNK, CHUNK)],
                                 dstv.at[b], sL[b])
                pltpu.async_copy(attr_h.at[pl.ds(ci * CHUNK, CHUNK)],
                                 attrv.at[b], sL[b])

        def wait_loads(t, b):
            @pl.when(valid(t))
            def _():
                ci = wid + NW * t
                pltpu.make_async_copy(dst_h.at[pl.ds(ci * CHUNK, CHUNK)],
                                      dstv.at[b], sL[b]).wait()
                pltpu.make_async_copy(attr_h.at[pl.ds(ci * CHUNK, CHUNK)],
                                      attrv.at[b], sL[b]).wait()

        def wait_scatters(t, b):
            @pl.when(valid(t))
            def _():
                pltpu.make_async_copy(attrv.at[b], accA.at[dstv.at[b]],
                                      sS[b]).wait()

        def step(t, b):
            bn = 1 - b
            wait_scatters(t - 1, bn)
            issue_loads(t + 1, bn)

            @pl.when(valid(t))
            def _():
                wait_loads(t, b)
                pltpu.async_copy(attrv.at[b], accA.at[dstv.at[b]], sS[b],
                                 add=True)

        issue_loads(0, 0)

        def ebody(j, carry):
            step(2 * j, 0)
            step(2 * j + 1, 1)
            return carry

        lax.fori_loop(0, ETRIPS // 2, ebody, 0)
        wait_scatters(ETRIPS - 1, 1)

        plsc.subcore_barrier()
        pltpu.sync_copy(accA.at[pl.ds(rs, ROWS_PER_SUB)],
                        accA_o.at[c, pl.ds(rs, ROWS_PER_SUB)])

    return sc_attr_scatter


@functools.cache
def _build_sc_geo_scatter():
    @functools.partial(
        pl.kernel,
        out_type=jax.ShapeDtypeStruct((NC, N_PAD, 8), jnp.float32),
        mesh=_mesh(),
        compiler_params=pltpu.CompilerParams(
            use_tc_tiling_on_sc=False, needs_layout_passes=False),
        scratch_types=[
            pltpu.VMEM((2, CHUNK), jnp.int32),
            pltpu.VMEM((2, CHUNK), jnp.int32),
            pltpu.VMEM((2, CHUNK, 8), jnp.float32),
            pltpu.SemaphoreType.DMA,
            pltpu.SemaphoreType.DMA,
            pltpu.SemaphoreType.DMA,
            pltpu.SemaphoreType.DMA,
            pltpu.SemaphoreType.DMA,
            pltpu.SemaphoreType.DMA,
            pltpu.VMEM_SHARED((N_PAD, 8), jnp.float32),
            pltpu.VMEM_SHARED((N_PAD, 8), jnp.float32),
        ],
    )
    def sc_geo_scatter(src_h, dst_h, geo_h, zG_h, accG_o,
                       srcv, dstv, geov, sL0, sL1, sG0, sG1, sS0, sS1, accG,
                       geoT):
        c = lax.axis_index("c")
        s = lax.axis_index("s")
        wid = c * NS + s
        rs = s * ROWS_PER_SUB
        sL = (sL0, sL1)
        sG = (sG0, sG1)
        sS = (sS0, sS1)

        pltpu.sync_copy(zG_h, accG.at[pl.ds(rs, ROWS_PER_SUB)])
        @pl.when(s < NS - 1)
        def _():
            pltpu.sync_copy(geo_h.at[pl.ds(rs, ROWS_PER_SUB)],
                            geoT.at[pl.ds(rs, ROWS_PER_SUB)])

        @pl.when(s == NS - 1)
        def _():
            pltpu.sync_copy(geo_h.at[pl.ds(rs, N - (NS - 1) * ROWS_PER_SUB)],
                            geoT.at[pl.ds(rs, N - (NS - 1) * ROWS_PER_SUB)])
        plsc.subcore_barrier()

        valid, issue_loads, wait_loads, wait_scatters = _pipe_helpers(
            wid, sL, sS, [accG], dstv, [geov],
            [(src_h, srcv), (dst_h, dstv)])

        def step(t, b):
            bn = 1 - b
            wait_scatters(t - 1, bn)
            issue_loads(t + 1, bn)

            @pl.when(valid(t))
            def _():
                wait_loads(t, b)
                pltpu.async_copy(geoT.at[srcv.at[b]], geov.at[b],
                                 sG[b]).wait()
                pltpu.async_copy(geov.at[b], accG.at[dstv.at[b]], sS[b],
                                 add=True)

        issue_loads(0, 0)

        def ebody(j, carry):
            step(2 * j, 0)
            step(2 * j + 1, 1)
            return carry

        lax.fori_loop(0, ETRIPS // 2, ebody, 0)
        wait_scatters(ETRIPS - 1, 1)

        plsc.subcore_barrier()
        pltpu.sync_copy(accG.at[pl.ds(rs, ROWS_PER_SUB)],
                        accG_o.at[c, pl.ds(rs, ROWS_PER_SUB)])

    return sc_geo_scatter


@functools.cache
def _build_sc_gather_masked():
    @functools.partial(
        pl.kernel,
        out_type=(
            jax.ShapeDtypeStruct((NMASK, D0), jnp.float32),
            jax.ShapeDtypeStruct((NMASK, D0), jnp.float32),
        ),
        mesh=_mesh(),
        scratch_types=[
            pltpu.VMEM((MCHUNK,), jnp.int32),
            pltpu.VMEM((MCHUNK, D0), jnp.float32),
            pltpu.VMEM((MCHUNK, D0), jnp.float32),
            pltpu.SemaphoreType.DMA,
            pltpu.SemaphoreType.DMA,
        ],
    )
    def sc_gather_masked(x_h, recon_h, mask_h, xpred_o, xtrue_o,
                         midx, rbuf, xbuf, sem1, sem2):
        c = lax.axis_index("c")
        s = lax.axis_index("s")
        wid = c * NS + s

        def mbody(t, carry):
            mc = wid + NW * t

            @pl.when(mc < NMCHUNKS)
            def _():
                moff = mc * MCHUNK
                pltpu.sync_copy(mask_h.at[pl.ds(moff, MCHUNK)], midx)
                cp1 = pltpu.async_copy(recon_h.at[midx], rbuf, sem1)
                cp2 = pltpu.async_copy(x_h.at[midx], xbuf, sem2)
                cp1.wait()
                pltpu.sync_copy(rbuf, xpred_o.at[pl.ds(moff, MCHUNK)])
                cp2.wait()
                pltpu.sync_copy(xbuf, xtrue_o.at[pl.ds(moff, MCHUNK)])
            return carry

        lax.fori_loop(0, MTRIPS, mbody, 0)

    return sc_gather_masked


def _sc_segment_sum(src, dst, attr, geo, mask32, zA, zG):
    accA, flagbuf = _build_sc_attr_scatter()(dst, attr, mask32, zA)
    accG = _build_sc_geo_scatter()(src, dst, geo, zG)
    return accA, accG, flagbuf


def _sc_gather_masked(x, recon, mask32):
    return _build_sc_gather_masked()(x, recon, mask32)


BN = 5000
GRID = N // BN


def _ln_k(z, g, b, eps=1e-5):
    m = jnp.mean(z, axis=-1, keepdims=True)
    v = jnp.mean((z - m) * (z - m), axis=-1, keepdims=True)
    return (z - m) * lax.rsqrt(v + eps) * g + b


def _tc_body(xb, aA0, aA1, aG0, aG1, flb, normb, tok,
             WxT, WgT, WaT, encb, encg, encbeta,
             npWT, npb, npg, npbeta, e2dWT,
             decWrT, decGT, decAT, decb, decg, decbeta,
             ns_o, rec_o):
    nrm = normb[...]
    ahg = (aG0[0] + aG1[0]) * nrm
    aha = (aA0[0] + aA1[0]) * nrm
    fl = flb[...]
    ox = xb[...] * (1.0 - fl) + tok[...] * fl
    dot = functools.partial(jnp.dot, preferred_element_type=jnp.float32)
    pre = (dot(ox, WxT[...]) + dot(ahg, WgT[...]) + dot(aha, WaT[...])
           + encb[...])
    enc = jnp.maximum(_ln_k(pre, encg[...], encbeta[...]), 0.0)
    ns_o[...] = _ln_k(dot(enc, npWT[...]) + npb[...], npg[...], npbeta[...])
    rep = dot(enc, e2dWT[...])
    dpre = (dot(rep, decWrT[...]) + dot(ahg, decGT[...]) + dot(aha, decAT[...])
            + decb[...])
    rec_o[...] = jnp.maximum(_ln_k(dpre, decg[...], decbeta[...]), 0.0)


def _row_spec(w):
    return pl.BlockSpec((BN, w), lambda i: (i, 0))


def _part_spec(w):
    return (pl.BlockSpec((1, BN, w), lambda i: (0, i, 0)),
            pl.BlockSpec((1, BN, w), lambda i: (1, i, 0)))


def _full_spec(a):
    r = len(a.shape)
    return pl.BlockSpec(a.shape, lambda i: (0,) * r)


def _tc_dense(x, accA, accG, flag, norm, tok, weights):
    aspec0, aspec1 = _part_spec(16)
    gspec0, gspec1 = _part_spec(8)
    wspecs = [_full_spec(w) for w in weights]
    return pl.pallas_call(
        _tc_body,
        grid=(GRID,),
        in_specs=[_row_spec(D0), aspec0, aspec1, gspec0, gspec1,
                  _row_spec(1), _row_spec(1), _full_spec(tok)] + wspecs,
        out_specs=[_row_spec(16), _row_spec(D0)],
        out_shape=[jax.ShapeDtypeStruct((N, 16), jnp.float32),
                   jax.ShapeDtypeStruct((N, D0), jnp.float32)],
    )(x, accA, accA, accG, accG, flag, norm, tok, *weights)


def kernel(x, edge_index, geometric, norm, distance, angle, feat,
           discrete_bin_edges, mask_nodes, enc_mask_token, enc_W, enc_b,
           enc_ln_g, enc_ln_b, e2d_W, np_W, np_b, np_ln_g, np_ln_b, dec_W,
           dec_b, dec_ln_g, dec_ln_b):
    src = edge_index[0].astype(jnp.int32)
    dst = edge_index[1].astype(jnp.int32)
    mask32 = mask_nodes.astype(jnp.int32)
    attr = jnp.concatenate(
        [distance[:, None], angle[:, None], feat, discrete_bin_edges], axis=1)
    zA = jnp.zeros((ROWS_PER_SUB, 16), jnp.float32)
    zG = jnp.zeros((ROWS_PER_SUB, 8), jnp.float32)
    accA, accG, flagbuf = _sc_segment_sum(src, dst, attr, geometric,
                                          mask32, zA, zG)
    flag = flagbuf.reshape(N_PAD, 1)

    weights = (
        enc_W[:, :D0].T,
        enc_W[:, D0:D0 + 8].T,
        enc_W[:, D0 + 8:D0 + 24].T,
        enc_b[None, :], enc_ln_g[None, :], enc_ln_b[None, :],
        np_W.T, np_b[None, :], np_ln_g[None, :], np_ln_b[None, :],
        e2d_W.T,
        dec_W[:, :D1].T,
        dec_W[:, D1:D1 + 8].T,
        dec_W[:, D1 + 8:D1 + 24].T,
        dec_b[None, :], dec_ln_g[None, :], dec_ln_b[None, :],
    )
    n_scores, recon = _tc_dense(x, accA, accG, flag, norm, enc_mask_token,
                                weights)
    x_pred, x_true = _sc_gather_masked(x, recon, mask32)
    return (x_pred, x_true, n_scores)
